# baseline (speedup 1.0000x reference)
import functools

import jax
import jax.numpy as jnp
from jax import lax
from jax.experimental import pallas as pl
from jax.experimental.pallas import tpu as pltpu
from jax.experimental.pallas import tpu_sc as plsc

_N = 10000
_E = 320000
_D = 128
_NC = 2
_NS = 16
_NW = _NC * _NS
_C = 128
_NCH = -(-_E // (_NW * _C))
_EP = _NW * _NCH * _C
_CH0 = 70
_CH1 = 88
_CHM = max(_CH0, _CH1)
_EP2 = _NS * (_CH0 + _CH1) * _C
_NP = 10240
_ZCH = _NP // (_NS * _C)
_RPT = _NP // _NS

_mesh = plsc.VectorSubcoreMesh(
    core_axis_name="c", subcore_axis_name="s", num_cores=_NC, num_subcores=_NS
)


def _sc_scatter_body(g_hbm, sd_hbm, out_hbm, ibuf, rows2, acc, gsem0, gsem1, isem0, isem1):
    c = lax.axis_index("c")
    s = lax.axis_index("s")
    w = s * _NC + c

    def zrow(i, carry):
        for j in range(_D // 16):
            rows2[0, i, pl.ds(j * 16, 16)] = jnp.zeros((16,), jnp.float32)
        return carry

    lax.fori_loop(0, _C, zrow, 0)
    for k in range(_ZCH):
        pltpu.sync_copy(rows2.at[0], acc.at[pl.ds((s * _ZCH + k) * _C, _C)])
    plsc.subcore_barrier()

    pltpu.sync_copy(sd_hbm.at[w, 0], ibuf.at[0])
    pltpu.sync_copy(sd_hbm.at[w, 1], ibuf.at[1])
    pltpu.async_copy(g_hbm.at[ibuf.at[0, 0]], rows2.at[0], gsem0)
    pltpu.async_copy(g_hbm.at[ibuf.at[1, 0]], rows2.at[1], gsem1)
    pltpu.async_copy(sd_hbm.at[w, 2], ibuf.at[2], isem0)
    pltpu.async_copy(sd_hbm.at[w, 3], ibuf.at[3], isem1)

    nch = jnp.where(c == 0, _CH0, _CH1)

    def one(j, u):
        rb = rows2.at[u]
        gsem = (gsem0, gsem1)[u]
        isem = (isem0, isem1)[u]
        q = lax.rem(j, 4)
        pltpu.make_async_copy(g_hbm.at[ibuf.at[q, 0]], rb, gsem).wait()
        pltpu.sync_copy(rb, acc.at[ibuf.at[q, 1]], add=True)

        @pl.when(j < nch - 2)
        def _():
            qn = lax.rem(j + 2, 4)
            pltpu.make_async_copy(sd_hbm.at[w, 0], ibuf.at[qn], isem).wait()
            pltpu.async_copy(g_hbm.at[ibuf.at[qn, 0]], rb, gsem)

        @pl.when(j < nch - 4)
        def _():
            pltpu.async_copy(sd_hbm.at[w, j + 4], ibuf.at[q], isem)

    def pair(t, carry):
        one(2 * t, 0)
        one(2 * t + 1, 1)
        return carry

    lax.fori_loop(0, nch // 2, pair, 0)
    plsc.subcore_barrier()
    pltpu.sync_copy(acc.at[pl.ds(s * _RPT, _RPT)], out_hbm.at[c, pl.ds(s * _RPT, _RPT)])


_sc_scatter = pl.kernel(
    _sc_scatter_body,
    out_type=jax.ShapeDtypeStruct((_NC, _NP, _D), jnp.float32),
    mesh=_mesh,
    scratch_types=[
        pltpu.VMEM((4, 2, _C), jnp.int32),
        pltpu.VMEM((2, _C, _D), jnp.float32),
        pltpu.VMEM_SHARED((_NP, _D), jnp.float32),
        pltpu.SemaphoreType.DMA,
        pltpu.SemaphoreType.DMA,
        pltpu.SemaphoreType.DMA,
        pltpu.SemaphoreType.DMA,
    ],
)


def _sc_degree_body(sd_hbm, out_hbm, sd_v, rows, acc, sem, isem):
    c = lax.axis_index("c")
    s = lax.axis_index("s")
    w = s * _NC + c

    idx_cp = pltpu.async_copy(sd_hbm.at[w], sd_v, isem)
    nch = jnp.where(c == 0, _CH0, _CH1)

    def fillv(val):
        def body(i, carry):
            for j in range(_D // 16):
                rows[i, pl.ds(j * 16, 16)] = jnp.full((16,), val, jnp.float32)
            return carry
        return body

    lax.fori_loop(0, _C, fillv(0.0), 0)
    for k in range(_ZCH):
        pltpu.sync_copy(rows, acc.at[pl.ds((s * _ZCH + k) * _C, _C)])
    lax.fori_loop(0, _C, fillv(1.0), 0)
    idx_cp.wait()
    plsc.subcore_barrier()

    pltpu.async_copy(rows, acc.at[sd_v.at[0, 1]], sem, add=True)

    def chunk(j, carry):
        @pl.when(j < nch - 1)
        def _():
            pltpu.async_copy(rows, acc.at[sd_v.at[j + 1, 1]], sem, add=True)

        pltpu.make_async_copy(rows, acc.at[sd_v.at[j, 1]], sem).wait()
        return carry

    lax.fori_loop(0, nch, chunk, 0)
    plsc.subcore_barrier()
    pltpu.sync_copy(acc.at[pl.ds(s * _RPT, _RPT)], out_hbm.at[c, pl.ds(s * _RPT, _RPT)])


_sc_degree = pl.kernel(
    _sc_degree_body,
    out_type=jax.ShapeDtypeStruct((_NC, _NP, _D), jnp.float32),
    mesh=_mesh,
    scratch_types=[
        pltpu.VMEM((_CHM, 2, _C), jnp.int32),
        pltpu.VMEM((_C, _D), jnp.float32),
        pltpu.VMEM_SHARED((_NP, _D), jnp.float32),
        pltpu.SemaphoreType.DMA,
        pltpu.SemaphoreType.DMA,
    ],
)


_RB = 2000
_GRID = (_N // _RB,)


def _tc_gin_body(deg_ref, x_ref, w_ref, g_ref, dinv_ref):
    d = 1.0 + deg_ref[0, :, 0:1] + deg_ref[1, :, 0:1]
    dinv = lax.rsqrt(d)
    dinv_ref[...] = dinv
    h = jnp.dot(x_ref[...], w_ref[...], preferred_element_type=jnp.float32)
    g_ref[...] = h * dinv


_tc_gin = pl.pallas_call(
    _tc_gin_body,
    grid=_GRID,
    in_specs=[
        pl.BlockSpec((2, _RB, _D), lambda i: (0, i, 0)),
        pl.BlockSpec((_RB, _D), lambda i: (i, 0)),
        pl.BlockSpec((_D, _D), lambda i: (0, 0)),
    ],
    out_specs=[
        pl.BlockSpec((_RB, _D), lambda i: (i, 0)),
        pl.BlockSpec((_RB, 1), lambda i: (i, 0)),
    ],
    out_shape=[
        jax.ShapeDtypeStruct((_N, _D), jnp.float32),
        jax.ShapeDtypeStruct((_N, 1), jnp.float32),
    ],
)


def _tc_layer_body(p_ref, g_ref, dinv_ref, b_ref, w_ref, gn_ref):
    dinv = dinv_ref[...]
    t = (p_ref[0] + p_ref[1] + g_ref[...]) * dinv + b_ref[...]
    act = jnp.where(t > 0, t, jnp.exp(jnp.minimum(t, 0.0)) - 1.0)
    h = jnp.dot(act, w_ref[...], preferred_element_type=jnp.float32)
    gn_ref[...] = h * dinv


_tc_layer = pl.pallas_call(
    _tc_layer_body,
    grid=_GRID,
    in_specs=[
        pl.BlockSpec((2, _RB, _D), lambda i: (0, i, 0)),
        pl.BlockSpec((_RB, _D), lambda i: (i, 0)),
        pl.BlockSpec((_RB, 1), lambda i: (i, 0)),
        pl.BlockSpec((1, _D), lambda i: (0, 0)),
        pl.BlockSpec((_D, _D), lambda i: (0, 0)),
    ],
    out_specs=pl.BlockSpec((_RB, _D), lambda i: (i, 0)),
    out_shape=jax.ShapeDtypeStruct((_N, _D), jnp.float32),
)


def _tc_final_body(p_ref, g_ref, dinv_ref, b_ref, o_ref):
    o_ref[...] = (p_ref[0] + p_ref[1] + g_ref[...]) * dinv_ref[...] + b_ref[...]


_tc_final = pl.pallas_call(
    _tc_final_body,
    grid=_GRID,
    in_specs=[
        pl.BlockSpec((2, _RB, _D), lambda i: (0, i, 0)),
        pl.BlockSpec((_RB, _D), lambda i: (i, 0)),
        pl.BlockSpec((_RB, 1), lambda i: (i, 0)),
        pl.BlockSpec((1, _D), lambda i: (0, 0)),
    ],
    out_specs=pl.BlockSpec((_RB, _D), lambda i: (i, 0)),
    out_shape=jax.ShapeDtypeStruct((_N, _D), jnp.float32),
)


def kernel(x, edge_index, W1, b1, W2, b2, W3, b3):
    src = edge_index[0]
    dst = edge_index[1]

    ecf = _NS * _CH0 * _C
    pad2 = _EP2 - _E
    srcp = jnp.concatenate([src, jnp.zeros((pad2,), jnp.int32)])
    dstp = jnp.concatenate([dst, jnp.full((pad2,), _N, jnp.int32)])
    def part(sp, dp, nch):
        g = jnp.stack([sp.reshape(_NS, nch, _C), dp.reshape(_NS, nch, _C)], axis=2)
        if nch < _CHM:
            g = jnp.concatenate(
                [g, jnp.zeros((_NS, _CHM - nch, 2, _C), jnp.int32)], axis=1
            )
        return g

    part0 = part(srcp[:ecf], dstp[:ecf], _CH0)
    part1 = part(srcp[ecf:], dstp[ecf:], _CH1)
    sd = jnp.stack([part0, part1], axis=1).reshape(_NW, _CHM, 2, _C)

    degp = _sc_degree(sd)
    g1, dinv = _tc_gin(degp, x, W1)
    p = _sc_scatter(g1, sd)
    g2 = _tc_layer(p, g1, dinv, b1.reshape(1, _D), W2)
    p = _sc_scatter(g2, sd)
    g3 = _tc_layer(p, g2, dinv, b2.reshape(1, _D), W3)
    p = _sc_scatter(g3, sd)
    return _tc_final(p, g3, dinv, b3.reshape(1, _D))


# speedup vs baseline: 1.1517x; 1.1517x over previous
import functools

import jax
import jax.numpy as jnp
from jax import lax
from jax.experimental import pallas as pl
from jax.experimental.pallas import tpu as pltpu
from jax.experimental.pallas import tpu_sc as plsc

_N = 10000
_E = 320000
_D = 128
_NC = 2
_NS = 16
_NW = _NC * _NS
_C = 128
_NCH = -(-_E // (_NW * _C))
_EP = _NW * _NCH * _C
_CH0 = 96
_CH1 = 62
_CHM = max(_CH0, _CH1)
_EP2 = _NS * (_CH0 + _CH1) * _C
_NP = 10240
_ZCH = _NP // (_NS * _C)
_RPT = _NP // _NS

_mesh = plsc.VectorSubcoreMesh(
    core_axis_name="c", subcore_axis_name="s", num_cores=_NC, num_subcores=_NS
)


def _sc_scatter_body(g_hbm, sd_hbm, out_hbm, ibuf, rows2, acc, gsem0, gsem1, isem0, isem1):
    c = lax.axis_index("c")
    s = lax.axis_index("s")
    w = s * _NC + c

    def zrow(i, carry):
        for j in range(_D // 16):
            rows2[0, i, pl.ds(j * 16, 16)] = jnp.zeros((16,), jnp.float32)
        return carry

    lax.fori_loop(0, _C, zrow, 0)
    for k in range(_ZCH):
        pltpu.sync_copy(rows2.at[0], acc.at[pl.ds((s * _ZCH + k) * _C, _C)])
    plsc.subcore_barrier()

    pltpu.sync_copy(sd_hbm.at[w, 0], ibuf.at[0])
    pltpu.sync_copy(sd_hbm.at[w, 1], ibuf.at[1])
    pltpu.async_copy(g_hbm.at[ibuf.at[0, 0]], rows2.at[0], gsem0)
    pltpu.async_copy(g_hbm.at[ibuf.at[1, 0]], rows2.at[1], gsem1)
    pltpu.async_copy(sd_hbm.at[w, 2], ibuf.at[2], isem0)
    pltpu.async_copy(sd_hbm.at[w, 3], ibuf.at[3], isem1)

    nch = jnp.where(c == 0, _CH0, _CH1)

    def one(j, u):
        rb = rows2.at[u]
        gsem = (gsem0, gsem1)[u]
        isem = (isem0, isem1)[u]
        q = lax.rem(j, 4)
        pltpu.make_async_copy(g_hbm.at[ibuf.at[q, 0]], rb, gsem).wait()
        pltpu.sync_copy(rb, acc.at[ibuf.at[q, 1]], add=True)

        @pl.when(j < nch - 2)
        def _():
            qn = lax.rem(j + 2, 4)
            pltpu.make_async_copy(sd_hbm.at[w, 0], ibuf.at[qn], isem).wait()
            pltpu.async_copy(g_hbm.at[ibuf.at[qn, 0]], rb, gsem)

        @pl.when(j < nch - 4)
        def _():
            pltpu.async_copy(sd_hbm.at[w, j + 4], ibuf.at[q], isem)

    def pair(t, carry):
        one(2 * t, 0)
        one(2 * t + 1, 1)
        return carry

    lax.fori_loop(0, nch // 2, pair, 0)
    plsc.subcore_barrier()
    pltpu.sync_copy(acc.at[pl.ds(s * _RPT, _RPT)], out_hbm.at[c, pl.ds(s * _RPT, _RPT)])


_sc_scatter = pl.kernel(
    _sc_scatter_body,
    out_type=jax.ShapeDtypeStruct((_NC, _NP, _D), jnp.float32),
    mesh=_mesh,
    scratch_types=[
        pltpu.VMEM((4, 2, _C), jnp.int32),
        pltpu.VMEM((2, _C, _D), jnp.float32),
        pltpu.VMEM_SHARED((_NP, _D), jnp.float32),
        pltpu.SemaphoreType.DMA,
        pltpu.SemaphoreType.DMA,
        pltpu.SemaphoreType.DMA,
        pltpu.SemaphoreType.DMA,
    ],
)


def _sc_degree_body(sd_hbm, out_hbm, sd_v, rows, acc, sem, isem):
    c = lax.axis_index("c")
    s = lax.axis_index("s")
    w = s * _NC + c

    idx_cp = pltpu.async_copy(sd_hbm.at[w], sd_v, isem)
    nch = jnp.where(c == 0, _CH0, _CH1)

    def fillv(val):
        def body(i, carry):
            for j in range(_D // 16):
                rows[i, pl.ds(j * 16, 16)] = jnp.full((16,), val, jnp.float32)
            return carry
        return body

    lax.fori_loop(0, _C, fillv(0.0), 0)
    for k in range(_ZCH):
        pltpu.sync_copy(rows, acc.at[pl.ds((s * _ZCH + k) * _C, _C)])
    lax.fori_loop(0, _C, fillv(1.0), 0)
    idx_cp.wait()
    plsc.subcore_barrier()

    pltpu.async_copy(rows, acc.at[sd_v.at[0, 1]], sem, add=True)

    def chunk(j, carry):
        @pl.when(j < nch - 1)
        def _():
            pltpu.async_copy(rows, acc.at[sd_v.at[j + 1, 1]], sem, add=True)

        pltpu.make_async_copy(rows, acc.at[sd_v.at[j, 1]], sem).wait()
        return carry

    lax.fori_loop(0, nch, chunk, 0)
    plsc.subcore_barrier()
    pltpu.sync_copy(acc.at[pl.ds(s * _RPT, _RPT)], out_hbm.at[c, pl.ds(s * _RPT, _RPT)])


_sc_degree = pl.kernel(
    _sc_degree_body,
    out_type=jax.ShapeDtypeStruct((_NC, _NP, _D), jnp.float32),
    mesh=_mesh,
    scratch_types=[
        pltpu.VMEM((_CHM, 2, _C), jnp.int32),
        pltpu.VMEM((_C, _D), jnp.float32),
        pltpu.VMEM_SHARED((_NP, _D), jnp.float32),
        pltpu.SemaphoreType.DMA,
        pltpu.SemaphoreType.DMA,
    ],
)


_RB = 2000
_GRID = (_N // _RB,)


def _tc_gin_body(deg_ref, x_ref, w_ref, g_ref, dinv_ref):
    d = 1.0 + deg_ref[0, :, 0:1] + deg_ref[1, :, 0:1]
    dinv = lax.rsqrt(d)
    dinv_ref[...] = dinv
    h = jnp.dot(x_ref[...], w_ref[...], preferred_element_type=jnp.float32)
    g_ref[...] = h * dinv


_tc_gin = pl.pallas_call(
    _tc_gin_body,
    grid=_GRID,
    in_specs=[
        pl.BlockSpec((2, _RB, _D), lambda i: (0, i, 0)),
        pl.BlockSpec((_RB, _D), lambda i: (i, 0)),
        pl.BlockSpec((_D, _D), lambda i: (0, 0)),
    ],
    out_specs=[
        pl.BlockSpec((_RB, _D), lambda i: (i, 0)),
        pl.BlockSpec((_RB, 1), lambda i: (i, 0)),
    ],
    out_shape=[
        jax.ShapeDtypeStruct((_N, _D), jnp.float32),
        jax.ShapeDtypeStruct((_N, 1), jnp.float32),
    ],
)


def _tc_layer_body(p_ref, g_ref, dinv_ref, b_ref, w_ref, gn_ref):
    dinv = dinv_ref[...]
    t = (p_ref[0] + p_ref[1] + g_ref[...]) * dinv + b_ref[...]
    act = jnp.where(t > 0, t, jnp.exp(jnp.minimum(t, 0.0)) - 1.0)
    h = jnp.dot(act, w_ref[...], preferred_element_type=jnp.float32)
    gn_ref[...] = h * dinv


_tc_layer = pl.pallas_call(
    _tc_layer_body,
    grid=_GRID,
    in_specs=[
        pl.BlockSpec((2, _RB, _D), lambda i: (0, i, 0)),
        pl.BlockSpec((_RB, _D), lambda i: (i, 0)),
        pl.BlockSpec((_RB, 1), lambda i: (i, 0)),
        pl.BlockSpec((1, _D), lambda i: (0, 0)),
        pl.BlockSpec((_D, _D), lambda i: (0, 0)),
    ],
    out_specs=pl.BlockSpec((_RB, _D), lambda i: (i, 0)),
    out_shape=jax.ShapeDtypeStruct((_N, _D), jnp.float32),
)


def _tc_final_body(p_ref, g_ref, dinv_ref, b_ref, o_ref):
    o_ref[...] = (p_ref[0] + p_ref[1] + g_ref[...]) * dinv_ref[...] + b_ref[...]


_tc_final = pl.pallas_call(
    _tc_final_body,
    grid=_GRID,
    in_specs=[
        pl.BlockSpec((2, _RB, _D), lambda i: (0, i, 0)),
        pl.BlockSpec((_RB, _D), lambda i: (i, 0)),
        pl.BlockSpec((_RB, 1), lambda i: (i, 0)),
        pl.BlockSpec((1, _D), lambda i: (0, 0)),
    ],
    out_specs=pl.BlockSpec((_RB, _D), lambda i: (i, 0)),
    out_shape=jax.ShapeDtypeStruct((_N, _D), jnp.float32),
)


def kernel(x, edge_index, W1, b1, W2, b2, W3, b3):
    src = edge_index[0]
    dst = edge_index[1]

    ecf = _NS * _CH0 * _C
    pad2 = _EP2 - _E
    srcp = jnp.concatenate([src, jnp.zeros((pad2,), jnp.int32)])
    dstp = jnp.concatenate([dst, jnp.full((pad2,), _N, jnp.int32)])
    def part(sp, dp, nch):
        g = jnp.stack([sp.reshape(_NS, nch, _C), dp.reshape(_NS, nch, _C)], axis=2)
        if nch < _CHM:
            g = jnp.concatenate(
                [g, jnp.zeros((_NS, _CHM - nch, 2, _C), jnp.int32)], axis=1
            )
        return g

    part0 = part(srcp[:ecf], dstp[:ecf], _CH0)
    part1 = part(srcp[ecf:], dstp[ecf:], _CH1)
    sd = jnp.stack([part0, part1], axis=1).reshape(_NW, _CHM, 2, _C)

    degp = _sc_degree(sd)
    g1, dinv = _tc_gin(degp, x, W1)
    p = _sc_scatter(g1, sd)
    g2 = _tc_layer(p, g1, dinv, b1.reshape(1, _D), W2)
    p = _sc_scatter(g2, sd)
    g3 = _tc_layer(p, g2, dinv, b2.reshape(1, _D), W3)
    p = _sc_scatter(g3, sd)
    return _tc_final(p, g3, dinv, b3.reshape(1, _D))
